# baseline (speedup 1.0000x reference)
import functools

import jax
import jax.numpy as jnp
from jax import lax
from jax.experimental import pallas as pl
from jax.experimental.pallas import tpu as pltpu
from jax.experimental.pallas import tpu_sc as plsc

_N = 10000
_E = 320000
_D = 128
_G = 64

_NC = 2
_NS = 16
_NW = _NC * _NS
_EPT = _E // _NW
_CHUNK = 120
_NCHUNK = 84
_EPTP = _NCHUNK * _CHUNK
_PADROW = _N
_ACCROWS = _NCHUNK * _CHUNK
_WROWS = 80
_NWCH = _N // _WROWS


_NBUF = 3


def _segment_sum_sc(x, src, dst):
    d = x.shape[1]
    mesh = plsc.VectorSubcoreMesh(core_axis_name="c", subcore_axis_name="s")

    @functools.partial(
        pl.kernel,
        out_type=jax.ShapeDtypeStruct((_NC, _N, d), jnp.float32),
        mesh=mesh,
        scratch_types=[
            pltpu.VMEM((_NBUF * _CHUNK,), jnp.int32),
            pltpu.VMEM((_NBUF * 8, _CHUNK), jnp.int32),
            [pltpu.VMEM((_CHUNK, d), jnp.float32) for _ in range(_NBUF)],
            pltpu.VMEM_SHARED((_ACCROWS, d), jnp.float32),
            [pltpu.SemaphoreType.DMA for _ in range(3 * _NBUF)],
        ],
    )
    def k(x_hbm, src_hbm, dst_hbm, out_hbm, sidx, didx, rows, acc, sems):
        c = lax.axis_index("c")
        s = lax.axis_index("s")
        wid = c * _NS + s
        isem = sems[:_NBUF]
        gsem = sems[_NBUF:2 * _NBUF]
        ssem = sems[2 * _NBUF:]

        zero = jnp.zeros((16,), jnp.float32)

        def zrow(r, carry):
            for j in range(d // 16):
                rows[2][r, pl.ds(j * 16, 16)] = zero
            return carry

        lax.fori_loop(0, _CHUNK, zrow, 0)

        def idx_start(j, b):
            pltpu.async_copy(
                src_hbm.at[wid, j, 0],
                sidx.at[pl.ds(b * _CHUNK, _CHUNK)], isem[b])
            pltpu.async_copy(dst_hbm.at[wid, j, 0], didx.at[8 * b], isem[b])

        def idx_wait(j, b):
            pltpu.make_async_copy(
                src_hbm.at[wid, j, 0],
                sidx.at[pl.ds(b * _CHUNK, _CHUNK)], isem[b]).wait()
            pltpu.make_async_copy(
                dst_hbm.at[wid, j, 0], didx.at[8 * b], isem[b]).wait()

        def gather_start(b):
            pltpu.async_copy(
                x_hbm.at[sidx.at[pl.ds(b * _CHUNK, _CHUNK)]], rows[b],
                gsem[b])

        def gather_wait(b):
            pltpu.make_async_copy(
                x_hbm.at[sidx.at[pl.ds(b * _CHUNK, _CHUNK)]], rows[b],
                gsem[b]).wait()

        def scatter_start(b):
            pltpu.async_copy(rows[b], acc.at[didx.at[8 * b]], ssem[b],
                             add=True)

        def scatter_wait(b):
            pltpu.make_async_copy(rows[b], acc.at[didx.at[8 * b]],
                                  ssem[b]).wait()

        def emit(j, b, swait_prev=True, idx_pf=True, g_pf=True):
            gather_wait(b)
            scatter_start(b)
            if swait_prev:
                scatter_wait((b + 2) % _NBUF)
            if idx_pf:
                idx_start(j + 2, (b + 2) % _NBUF)
            if g_pf:
                nb = (b + 1) % _NBUF
                idx_wait(j + 1, nb)
                gather_start(nb)

        idx_start(0, 0)
        idx_start(1, 1)
        idx_wait(0, 0)
        gather_start(0)

        nz = jnp.where(s < _NCHUNK - _NS * (_NCHUNK // _NS),
                       _NCHUNK // _NS + 1, _NCHUNK // _NS)

        def zloop(i, carry):
            pltpu.sync_copy(
                rows[2], acc.at[pl.ds((s + i * _NS) * _CHUNK, _CHUNK)])
            return carry

        lax.fori_loop(0, nz, zloop, 0)
        plsc.subcore_barrier()

        emit(0, 0, swait_prev=False)
        emit(1, 1)

        def body(kk, carry):
            j0 = 2 + 3 * kk
            for u in range(3):
                emit(j0 + u, (2 + u) % _NBUF)
            return carry

        lax.fori_loop(0, 26, body, 0)

        emit(_NCHUNK - 4, (_NCHUNK - 4) % _NBUF)
        emit(_NCHUNK - 3, (_NCHUNK - 3) % _NBUF)
        emit(_NCHUNK - 2, (_NCHUNK - 2) % _NBUF, idx_pf=False)
        emit(_NCHUNK - 1, (_NCHUNK - 1) % _NBUF, idx_pf=False, g_pf=False)
        scatter_wait((_NCHUNK - 1) % _NBUF)
        plsc.subcore_barrier()

        nw = jnp.where(s < _NWCH - _NS * (_NWCH // _NS),
                       _NWCH // _NS + 1, _NWCH // _NS)

        def wloop(i, carry):
            r0 = (s + i * _NS) * _WROWS
            pltpu.sync_copy(
                acc.at[pl.ds(r0, _WROWS)],
                out_hbm.at[c, pl.ds(r0, _WROWS)],
            )
            return carry

        lax.fori_loop(0, nw, wloop, 0)

    return k(x, src, dst)


_BLK = 1000


def _root_matmul_tc(x, w_root, b):
    d = x.shape[1]
    h = w_root.shape[1]

    def body(x_ref, wr_ref, b_ref, o_ref):
        o_ref[...] = jnp.dot(
            x_ref[...], wr_ref[...],
            preferred_element_type=jnp.float32) + b_ref[...]

    return pl.pallas_call(
        body,
        grid=(_N // _BLK,),
        in_specs=[
            pl.BlockSpec((_BLK, d), lambda i: (i, 0)),
            pl.BlockSpec((d, h), lambda i: (0, 0)),
            pl.BlockSpec((1, h), lambda i: (0, 0)),
        ],
        out_specs=pl.BlockSpec((_BLK, h), lambda i: (i, 0)),
        out_shape=jax.ShapeDtypeStruct((_N, h), jnp.float32),
    )(x, w_root, b.reshape(1, h))


def _combine_tc(r, p, w_nei):
    d = p.shape[2]
    h = w_nei.shape[1]

    def body(r_ref, p0_ref, p1_ref, wn_ref, o_ref):
        agg = p0_ref[...] + p1_ref[...]
        acc = r_ref[...] + jnp.dot(agg, wn_ref[...],
                                   preferred_element_type=jnp.float32)
        o_ref[...] = jnp.maximum(acc, 0.0)

    return pl.pallas_call(
        body,
        grid=(_N // _BLK,),
        in_specs=[
            pl.BlockSpec((_BLK, h), lambda i: (i, 0)),
            pl.BlockSpec((_BLK, d), lambda i: (i, 0)),
            pl.BlockSpec((_BLK, d), lambda i: (i, 0)),
            pl.BlockSpec((d, h), lambda i: (0, 0)),
        ],
        out_specs=pl.BlockSpec((_BLK, h), lambda i: (i, 0)),
        out_shape=jax.ShapeDtypeStruct((_N, h), jnp.float32),
    )(r, p[0], p[1], w_nei)


def _combine2_pool_tc(r, p, w_nei, batch):
    d = p.shape[2]
    h = w_nei.shape[1]
    nblk = _N // _BLK

    def body(r_ref, p0_ref, p1_ref, wn_ref, bat_ref, o_ref, acc_ref, cnt_ref):
        i = pl.program_id(0)
        agg = p0_ref[...] + p1_ref[...]
        hh = r_ref[...] + jnp.dot(agg, wn_ref[...],
                                  preferred_element_type=jnp.float32)
        hh = jnp.maximum(hh, 0.0)

        onehot = (bat_ref[...] ==
                  lax.broadcasted_iota(jnp.int32, (_BLK, _G), 1)
                  ).astype(jnp.float32)
        part = lax.dot_general(onehot, hh, (((0,), (0,)), ((), ())),
                               preferred_element_type=jnp.float32)
        ones = jnp.ones((_BLK, h), jnp.float32)
        pcnt = lax.dot_general(onehot, ones, (((0,), (0,)), ((), ())),
                               preferred_element_type=jnp.float32)

        @pl.when(i == 0)
        def _():
            acc_ref[...] = jnp.zeros_like(acc_ref)
            cnt_ref[...] = jnp.zeros_like(cnt_ref)

        acc_ref[...] += part
        cnt_ref[...] += pcnt

        @pl.when(i == nblk - 1)
        def _():
            o_ref[...] = acc_ref[...] / jnp.maximum(cnt_ref[...], 1.0)

    return pl.pallas_call(
        body,
        grid=(nblk,),
        in_specs=[
            pl.BlockSpec((_BLK, h), lambda i: (i, 0)),
            pl.BlockSpec((_BLK, d), lambda i: (i, 0)),
            pl.BlockSpec((_BLK, d), lambda i: (i, 0)),
            pl.BlockSpec((d, h), lambda i: (0, 0)),
            pl.BlockSpec((_BLK, 1), lambda i: (i, 0)),
        ],
        out_specs=pl.BlockSpec((_G, h), lambda i: (0, 0)),
        out_shape=jax.ShapeDtypeStruct((_G, h), jnp.float32),
        scratch_shapes=[
            pltpu.VMEM((_G, h), jnp.float32),
            pltpu.VMEM((_G, h), jnp.float32),
        ],
    )(r, p[0], p[1], w_nei, batch.reshape(_N, 1))


def kernel(x, edge_index, batch, W1_root, W1_nei, b1, W2_root, W2_nei, b2):
    pad = _EPTP - _EPT
    src = jnp.pad(edge_index[0].reshape(_NW, _EPT), ((0, 0), (0, pad))
                  ).reshape(_NW, _NCHUNK, 1, _CHUNK)
    dst = jnp.pad(edge_index[1].reshape(_NW, _EPT), ((0, 0), (0, pad)),
                  constant_values=_PADROW).reshape(_NW, _NCHUNK, 1, _CHUNK)
    p1 = _segment_sum_sc(x, src, dst)
    r1 = _root_matmul_tc(x, W1_root, b1)
    h = _combine_tc(r1, p1, W1_nei)
    p2 = _segment_sum_sc(h, src, dst)
    r2 = _root_matmul_tc(h, W2_root, b2)
    return _combine2_pool_tc(r2, p2, W2_nei, batch)


# speedup vs baseline: 1.3614x; 1.3614x over previous
import functools

import jax
import jax.numpy as jnp
from jax import lax
from jax.experimental import pallas as pl
from jax.experimental.pallas import tpu as pltpu
from jax.experimental.pallas import tpu_sc as plsc

_N = 10000
_E = 320000
_D = 128
_G = 64

_NC = 2
_NS = 16
_NW = _NC * _NS
_EPT = _E // _NW
_CHUNK = 64
_NCHUNK = 157
_EPTP = _NCHUNK * _CHUNK
_PADROW = _N
_ACCROWS = _NCHUNK * _CHUNK
_WROWS = 80
_NWCH = _N // _WROWS


_NBUF = 5


def _segment_sum_sc(x, src, dst):
    d = x.shape[1]
    mesh = plsc.VectorSubcoreMesh(core_axis_name="c", subcore_axis_name="s")

    @functools.partial(
        pl.kernel,
        out_type=jax.ShapeDtypeStruct((_NC, _N, d), jnp.float32),
        mesh=mesh,
        scratch_types=[
            pltpu.VMEM((_NBUF * _CHUNK,), jnp.int32),
            pltpu.VMEM((_NBUF * 8, _CHUNK), jnp.int32),
            [pltpu.VMEM((_CHUNK, d), jnp.float32) for _ in range(_NBUF)],
            pltpu.VMEM_SHARED((_ACCROWS, d), jnp.float32),
            [pltpu.SemaphoreType.DMA for _ in range(3 * _NBUF)],
        ],
    )
    def k(x_hbm, src_hbm, dst_hbm, out_hbm, sidx, didx, rows, acc, sems):
        c = lax.axis_index("c")
        s = lax.axis_index("s")
        wid = c * _NS + s
        isem = sems[:_NBUF]
        gsem = sems[_NBUF:2 * _NBUF]
        ssem = sems[2 * _NBUF:]

        zero = jnp.zeros((16,), jnp.float32)

        def zrow(r, carry):
            for j in range(d // 16):
                rows[_NBUF - 1][r, pl.ds(j * 16, 16)] = zero
            return carry

        lax.fori_loop(0, _CHUNK, zrow, 0)

        def idx_start(j, b):
            pltpu.async_copy(
                src_hbm.at[wid, j, 0],
                sidx.at[pl.ds(b * _CHUNK, _CHUNK)], isem[b])
            pltpu.async_copy(dst_hbm.at[wid, j, 0], didx.at[8 * b], isem[b])

        def idx_wait(j, b):
            pltpu.make_async_copy(
                src_hbm.at[wid, j, 0],
                sidx.at[pl.ds(b * _CHUNK, _CHUNK)], isem[b]).wait()
            pltpu.make_async_copy(
                dst_hbm.at[wid, j, 0], didx.at[8 * b], isem[b]).wait()

        def gather_start(b):
            pltpu.async_copy(
                x_hbm.at[sidx.at[pl.ds(b * _CHUNK, _CHUNK)]], rows[b],
                gsem[b])

        def gather_wait(b):
            pltpu.make_async_copy(
                x_hbm.at[sidx.at[pl.ds(b * _CHUNK, _CHUNK)]], rows[b],
                gsem[b]).wait()

        def scatter_start(b):
            pltpu.async_copy(rows[b], acc.at[didx.at[8 * b]], ssem[b],
                             add=True)

        def scatter_wait(b):
            pltpu.make_async_copy(rows[b], acc.at[didx.at[8 * b]],
                                  ssem[b]).wait()

        def emit(j, b, swait_prev=True, idx_pf=True, g_pf=True):
            gather_wait(b)
            scatter_start(b)
            if swait_prev:
                scatter_wait((b - 1) % _NBUF)
            if idx_pf:
                idx_start(j + _NBUF - 1, (b - 1) % _NBUF)
            if g_pf:
                nb = (b - 2) % _NBUF
                idx_wait(j + _NBUF - 2, nb)
                gather_start(nb)

        for b in range(_NBUF - 1):
            idx_start(b, b)
        for b in range(_NBUF - 2):
            idx_wait(b, b)
            gather_start(b)

        nz = jnp.where(s < _NCHUNK - _NS * (_NCHUNK // _NS),
                       _NCHUNK // _NS + 1, _NCHUNK // _NS)

        def zloop(i, carry):
            pltpu.sync_copy(
                rows[_NBUF - 1],
                acc.at[pl.ds((s + i * _NS) * _CHUNK, _CHUNK)])
            return carry

        lax.fori_loop(0, nz, zloop, 0)
        plsc.subcore_barrier()

        emit(0, 0, swait_prev=False)
        emit(1, 1)
        emit(2, 2)

        def body(kk, carry):
            j0 = 3 + _NBUF * kk
            for u in range(_NBUF):
                emit(j0 + u, (3 + u) % _NBUF)
            return carry

        lax.fori_loop(0, (_NCHUNK - _NBUF - 3 + 1) // _NBUF, body, 0)

        emit(_NCHUNK - 4, (_NCHUNK - 4) % _NBUF, idx_pf=False)
        emit(_NCHUNK - 3, (_NCHUNK - 3) % _NBUF, idx_pf=False, g_pf=False)
        emit(_NCHUNK - 2, (_NCHUNK - 2) % _NBUF, idx_pf=False, g_pf=False)
        emit(_NCHUNK - 1, (_NCHUNK - 1) % _NBUF, idx_pf=False, g_pf=False)
        scatter_wait((_NCHUNK - 1) % _NBUF)
        plsc.subcore_barrier()

        nw = jnp.where(s < _NWCH - _NS * (_NWCH // _NS),
                       _NWCH // _NS + 1, _NWCH // _NS)

        def wloop(i, carry):
            r0 = (s + i * _NS) * _WROWS
            pltpu.sync_copy(
                acc.at[pl.ds(r0, _WROWS)],
                out_hbm.at[c, pl.ds(r0, _WROWS)],
            )
            return carry

        lax.fori_loop(0, nw, wloop, 0)

    return k(x, src, dst)


_BLK = 1000


def _root_matmul_tc(x, w_root, b):
    d = x.shape[1]
    h = w_root.shape[1]

    def body(x_ref, wr_ref, b_ref, o_ref):
        o_ref[...] = jnp.dot(
            x_ref[...], wr_ref[...],
            preferred_element_type=jnp.float32) + b_ref[...]

    return pl.pallas_call(
        body,
        grid=(_N // _BLK,),
        in_specs=[
            pl.BlockSpec((_BLK, d), lambda i: (i, 0)),
            pl.BlockSpec((d, h), lambda i: (0, 0)),
            pl.BlockSpec((1, h), lambda i: (0, 0)),
        ],
        out_specs=pl.BlockSpec((_BLK, h), lambda i: (i, 0)),
        out_shape=jax.ShapeDtypeStruct((_N, h), jnp.float32),
    )(x, w_root, b.reshape(1, h))


def _combine_tc(r, p, w_nei):
    d = p.shape[2]
    h = w_nei.shape[1]

    def body(r_ref, p0_ref, p1_ref, wn_ref, o_ref):
        agg = p0_ref[...] + p1_ref[...]
        acc = r_ref[...] + jnp.dot(agg, wn_ref[...],
                                   preferred_element_type=jnp.float32)
        o_ref[...] = jnp.maximum(acc, 0.0)

    return pl.pallas_call(
        body,
        grid=(_N // _BLK,),
        in_specs=[
            pl.BlockSpec((_BLK, h), lambda i: (i, 0)),
            pl.BlockSpec((_BLK, d), lambda i: (i, 0)),
            pl.BlockSpec((_BLK, d), lambda i: (i, 0)),
            pl.BlockSpec((d, h), lambda i: (0, 0)),
        ],
        out_specs=pl.BlockSpec((_BLK, h), lambda i: (i, 0)),
        out_shape=jax.ShapeDtypeStruct((_N, h), jnp.float32),
    )(r, p[0], p[1], w_nei)


def _combine2_pool_tc(r, p, w_nei, batch):
    d = p.shape[2]
    h = w_nei.shape[1]
    nblk = _N // _BLK

    def body(r_ref, p0_ref, p1_ref, wn_ref, bat_ref, o_ref, acc_ref, cnt_ref):
        i = pl.program_id(0)
        agg = p0_ref[...] + p1_ref[...]
        hh = r_ref[...] + jnp.dot(agg, wn_ref[...],
                                  preferred_element_type=jnp.float32)
        hh = jnp.maximum(hh, 0.0)

        onehot = (bat_ref[...] ==
                  lax.broadcasted_iota(jnp.int32, (_BLK, _G), 1)
                  ).astype(jnp.float32)
        part = lax.dot_general(onehot, hh, (((0,), (0,)), ((), ())),
                               preferred_element_type=jnp.float32)
        ones = jnp.ones((_BLK, h), jnp.float32)
        pcnt = lax.dot_general(onehot, ones, (((0,), (0,)), ((), ())),
                               preferred_element_type=jnp.float32)

        @pl.when(i == 0)
        def _():
            acc_ref[...] = jnp.zeros_like(acc_ref)
            cnt_ref[...] = jnp.zeros_like(cnt_ref)

        acc_ref[...] += part
        cnt_ref[...] += pcnt

        @pl.when(i == nblk - 1)
        def _():
            o_ref[...] = acc_ref[...] / jnp.maximum(cnt_ref[...], 1.0)

    return pl.pallas_call(
        body,
        grid=(nblk,),
        in_specs=[
            pl.BlockSpec((_BLK, h), lambda i: (i, 0)),
            pl.BlockSpec((_BLK, d), lambda i: (i, 0)),
            pl.BlockSpec((_BLK, d), lambda i: (i, 0)),
            pl.BlockSpec((d, h), lambda i: (0, 0)),
            pl.BlockSpec((_BLK, 1), lambda i: (i, 0)),
        ],
        out_specs=pl.BlockSpec((_G, h), lambda i: (0, 0)),
        out_shape=jax.ShapeDtypeStruct((_G, h), jnp.float32),
        scratch_shapes=[
            pltpu.VMEM((_G, h), jnp.float32),
            pltpu.VMEM((_G, h), jnp.float32),
        ],
    )(r, p[0], p[1], w_nei, batch.reshape(_N, 1))


def kernel(x, edge_index, batch, W1_root, W1_nei, b1, W2_root, W2_nei, b2):
    pad = _EPTP - _EPT
    src = jnp.pad(edge_index[0].reshape(_NW, _EPT), ((0, 0), (0, pad))
                  ).reshape(_NW, _NCHUNK, 1, _CHUNK)
    dst = jnp.pad(edge_index[1].reshape(_NW, _EPT), ((0, 0), (0, pad)),
                  constant_values=_PADROW).reshape(_NW, _NCHUNK, 1, _CHUNK)
    p1 = _segment_sum_sc(x, src, dst)
    r1 = _root_matmul_tc(x, W1_root, b1)
    h = _combine_tc(r1, p1, W1_nei)
    p2 = _segment_sum_sc(h, src, dst)
    r2 = _root_matmul_tc(h, W2_root, b2)
    return _combine2_pool_tc(r2, p2, W2_nei, batch)


# speedup vs baseline: 1.8970x; 1.3935x over previous
import functools

import jax
import jax.numpy as jnp
from jax import lax
from jax.experimental import pallas as pl
from jax.experimental.pallas import tpu as pltpu
from jax.experimental.pallas import tpu_sc as plsc

_N = 10000
_E = 320000
_D = 128
_G = 64

_NC = 2
_NS = 16
_NW = _NC * _NS
_EPT = _E // _NW
_CHUNK = 80
_NCHUNK = _EPT // _CHUNK
_WROWS = 80
_NWCH = _N // _WROWS


_NBUF = 4
_NIDX = 8


def _segment_sum_sc(x, src, dst):
    d = x.shape[1]
    mesh = plsc.VectorSubcoreMesh(core_axis_name="c", subcore_axis_name="s")

    @functools.partial(
        pl.kernel,
        out_type=jax.ShapeDtypeStruct((_NC, _N, d), jnp.float32),
        mesh=mesh,
        scratch_types=[
            pltpu.VMEM((_NIDX * _CHUNK,), jnp.int32),
            pltpu.VMEM((_NIDX * 8, _CHUNK), jnp.int32),
            [pltpu.VMEM((_CHUNK, d), jnp.float32) for _ in range(_NBUF)],
            pltpu.VMEM_SHARED((_N, d), jnp.float32),
            [pltpu.SemaphoreType.DMA for _ in range(_NIDX + 2 * _NBUF)],
        ],
    )
    def k(x_hbm, src_hbm, dst_hbm, out_hbm, sidx, didx, rows, acc, sems):
        c = lax.axis_index("c")
        s = lax.axis_index("s")
        wid = c * _NS + s
        isem = sems[:_NIDX]
        gsem = sems[_NIDX:_NIDX + _NBUF]
        ssem = sems[_NIDX + _NBUF:]

        zero = jnp.zeros((16,), jnp.float32)

        def zrow(r, carry):
            for j in range(d // 16):
                rows[2][r, pl.ds(j * 16, 16)] = zero
            return carry

        lax.fori_loop(0, _CHUNK, zrow, 0)

        def idx_start(j, ib):
            pltpu.async_copy(
                src_hbm.at[wid, j, 0],
                sidx.at[pl.ds(ib * _CHUNK, _CHUNK)], isem[ib])
            pltpu.async_copy(dst_hbm.at[wid, j, 0], didx.at[8 * ib],
                             isem[ib])

        def idx_wait(j, ib):
            pltpu.make_async_copy(
                src_hbm.at[wid, j, 0],
                sidx.at[pl.ds(ib * _CHUNK, _CHUNK)], isem[ib]).wait()
            pltpu.make_async_copy(
                dst_hbm.at[wid, j, 0], didx.at[8 * ib], isem[ib]).wait()

        def gather_start(b, ib):
            pltpu.async_copy(
                x_hbm.at[sidx.at[pl.ds(ib * _CHUNK, _CHUNK)]], rows[b],
                gsem[b])

        def gather_wait(b, ib):
            pltpu.make_async_copy(
                x_hbm.at[sidx.at[pl.ds(ib * _CHUNK, _CHUNK)]], rows[b],
                gsem[b]).wait()

        def scatter_start(b, ib):
            pltpu.async_copy(rows[b], acc.at[didx.at[8 * ib]], ssem[b],
                             add=True)

        def scatter_wait(b, ib):
            pltpu.make_async_copy(rows[b], acc.at[didx.at[8 * ib]],
                                  ssem[b]).wait()

        def emit(j, b, ib, swait_prev=True, idx_pf=True, g_pf=True):
            gather_wait(b, ib)
            scatter_start(b, ib)
            if swait_prev:
                scatter_wait((b + 2) % _NBUF, (ib + _NIDX - 2) % _NIDX)
            if idx_pf:
                idx_start(j + 3, (ib + 3) % _NIDX)
            if g_pf:
                idx_wait(j + 2, (ib + 2) % _NIDX)
                gather_start((b + 2) % _NBUF, (ib + 2) % _NIDX)

        idx_start(0, 0)
        idx_start(1, 1)
        idx_start(2, 2)
        idx_wait(0, 0)
        gather_start(0, 0)
        idx_wait(1, 1)
        gather_start(1, 1)

        nz = jnp.where(s < _NWCH - _NS * (_NWCH // _NS),
                       _NWCH // _NS + 1, _NWCH // _NS)

        def zloop(i, carry):
            pltpu.sync_copy(
                rows[2], acc.at[pl.ds((s + i * _NS) * _CHUNK, _CHUNK)])
            return carry

        lax.fori_loop(0, nz, zloop, 0)
        plsc.subcore_barrier()

        emit(0, 0, 0, swait_prev=False)
        emit(1, 1, 1, swait_prev=False)

        def body(kk, carry):
            j0 = 2 + 8 * kk
            for u in range(8):
                emit(j0 + u, (2 + u) % _NBUF, (2 + u) % _NIDX)
            return carry

        lax.fori_loop(0, (_NCHUNK - 5) // 8, body, 0)

        emit(_NCHUNK - 3, (_NCHUNK - 3) % _NBUF, (_NCHUNK - 3) % _NIDX,
             idx_pf=False)
        emit(_NCHUNK - 2, (_NCHUNK - 2) % _NBUF, (_NCHUNK - 2) % _NIDX,
             idx_pf=False, g_pf=False)
        emit(_NCHUNK - 1, (_NCHUNK - 1) % _NBUF, (_NCHUNK - 1) % _NIDX,
             idx_pf=False, g_pf=False)
        scatter_wait((_NCHUNK - 2) % _NBUF, (_NCHUNK - 2) % _NIDX)
        scatter_wait((_NCHUNK - 1) % _NBUF, (_NCHUNK - 1) % _NIDX)
        plsc.subcore_barrier()

        def wloop(i, carry):
            r0 = (s + i * _NS) * _WROWS
            pltpu.sync_copy(
                acc.at[pl.ds(r0, _WROWS)],
                out_hbm.at[c, pl.ds(r0, _WROWS)],
            )
            return carry

        lax.fori_loop(0, nz, wloop, 0)

    return k(x, src, dst)


_BLK = 1000


def _root_matmul_tc(x, w_root, b):
    d = x.shape[1]
    h = w_root.shape[1]

    def body(x_ref, wr_ref, b_ref, o_ref):
        o_ref[...] = jnp.dot(
            x_ref[...], wr_ref[...],
            preferred_element_type=jnp.float32) + b_ref[...]

    return pl.pallas_call(
        body,
        grid=(_N // _BLK,),
        in_specs=[
            pl.BlockSpec((_BLK, d), lambda i: (i, 0)),
            pl.BlockSpec((d, h), lambda i: (0, 0)),
            pl.BlockSpec((1, h), lambda i: (0, 0)),
        ],
        out_specs=pl.BlockSpec((_BLK, h), lambda i: (i, 0)),
        out_shape=jax.ShapeDtypeStruct((_N, h), jnp.float32),
    )(x, w_root, b.reshape(1, h))


def _combine_tc(r, p, w_nei):
    d = p.shape[2]
    h = w_nei.shape[1]

    def body(r_ref, p0_ref, p1_ref, wn_ref, o_ref):
        agg = p0_ref[...] + p1_ref[...]
        acc = r_ref[...] + jnp.dot(agg, wn_ref[...],
                                   preferred_element_type=jnp.float32)
        o_ref[...] = jnp.maximum(acc, 0.0)

    return pl.pallas_call(
        body,
        grid=(_N // _BLK,),
        in_specs=[
            pl.BlockSpec((_BLK, h), lambda i: (i, 0)),
            pl.BlockSpec((_BLK, d), lambda i: (i, 0)),
            pl.BlockSpec((_BLK, d), lambda i: (i, 0)),
            pl.BlockSpec((d, h), lambda i: (0, 0)),
        ],
        out_specs=pl.BlockSpec((_BLK, h), lambda i: (i, 0)),
        out_shape=jax.ShapeDtypeStruct((_N, h), jnp.float32),
    )(r, p[0], p[1], w_nei)


def _combine2_pool_tc(r, p, w_nei, batch):
    d = p.shape[2]
    h = w_nei.shape[1]
    nblk = _N // _BLK

    def body(r_ref, p0_ref, p1_ref, wn_ref, bat_ref, o_ref, acc_ref, cnt_ref):
        i = pl.program_id(0)
        agg = p0_ref[...] + p1_ref[...]
        hh = r_ref[...] + jnp.dot(agg, wn_ref[...],
                                  preferred_element_type=jnp.float32)
        hh = jnp.maximum(hh, 0.0)

        onehot = (bat_ref[...] ==
                  lax.broadcasted_iota(jnp.int32, (_BLK, _G), 1)
                  ).astype(jnp.float32)
        part = lax.dot_general(onehot, hh, (((0,), (0,)), ((), ())),
                               preferred_element_type=jnp.float32)
        ones = jnp.ones((_BLK, h), jnp.float32)
        pcnt = lax.dot_general(onehot, ones, (((0,), (0,)), ((), ())),
                               preferred_element_type=jnp.float32)

        @pl.when(i == 0)
        def _():
            acc_ref[...] = jnp.zeros_like(acc_ref)
            cnt_ref[...] = jnp.zeros_like(cnt_ref)

        acc_ref[...] += part
        cnt_ref[...] += pcnt

        @pl.when(i == nblk - 1)
        def _():
            o_ref[...] = acc_ref[...] / jnp.maximum(cnt_ref[...], 1.0)

    return pl.pallas_call(
        body,
        grid=(nblk,),
        in_specs=[
            pl.BlockSpec((_BLK, h), lambda i: (i, 0)),
            pl.BlockSpec((_BLK, d), lambda i: (i, 0)),
            pl.BlockSpec((_BLK, d), lambda i: (i, 0)),
            pl.BlockSpec((d, h), lambda i: (0, 0)),
            pl.BlockSpec((_BLK, 1), lambda i: (i, 0)),
        ],
        out_specs=pl.BlockSpec((_G, h), lambda i: (0, 0)),
        out_shape=jax.ShapeDtypeStruct((_G, h), jnp.float32),
        scratch_shapes=[
            pltpu.VMEM((_G, h), jnp.float32),
            pltpu.VMEM((_G, h), jnp.float32),
        ],
    )(r, p[0], p[1], w_nei, batch.reshape(_N, 1))


def kernel(x, edge_index, batch, W1_root, W1_nei, b1, W2_root, W2_nei, b2):
    src = edge_index[0].reshape(_NW, _NCHUNK, 1, _CHUNK)
    dst = edge_index[1].reshape(_NW, _NCHUNK, 1, _CHUNK)
    p1 = _segment_sum_sc(x, src, dst)
    r1 = _root_matmul_tc(x, W1_root, b1)
    h = _combine_tc(r1, p1, W1_nei)
    p2 = _segment_sum_sc(h, src, dst)
    r2 = _root_matmul_tc(h, W2_root, b2)
    return _combine2_pool_tc(r2, p2, W2_nei, batch)


# speedup vs baseline: 2.0448x; 1.0779x over previous
import functools

import jax
import jax.numpy as jnp
from jax import lax
from jax.experimental import pallas as pl
from jax.experimental.pallas import tpu as pltpu
from jax.experimental.pallas import tpu_sc as plsc

_N = 10000
_E = 320000
_D = 128
_G = 64

_NC = 2
_NS = 16
_NW = _NC * _NS
_EPT = _E // _NW
_CHUNK = 80
_NCHUNK = _EPT // _CHUNK
_WROWS = 80
_NWCH = _N // _WROWS


_NBUF = 4
_NIDX = 8


def _segment_sum_sc(x, src, dst):
    d = x.shape[1]
    mesh = plsc.VectorSubcoreMesh(core_axis_name="c", subcore_axis_name="s")

    @functools.partial(
        pl.kernel,
        out_type=jax.ShapeDtypeStruct((_NC, _N, d), jnp.float32),
        mesh=mesh,
        scratch_types=[
            pltpu.VMEM((_NIDX * _CHUNK,), jnp.int32),
            pltpu.VMEM((_NIDX * 8, _CHUNK), jnp.int32),
            [pltpu.VMEM((_CHUNK, d), jnp.float32) for _ in range(_NBUF)],
            pltpu.VMEM_SHARED((_N, d), jnp.float32),
            [pltpu.SemaphoreType.DMA for _ in range(_NIDX + 2 * _NBUF)],
        ],
    )
    def k(x_hbm, src_hbm, dst_hbm, out_hbm, sidx, didx, rows, acc, sems):
        c = lax.axis_index("c")
        s = lax.axis_index("s")
        wid = c * _NS + s
        isem = sems[:_NIDX]
        gsem = sems[_NIDX:_NIDX + _NBUF]
        ssem = sems[_NIDX + _NBUF:]

        zero = jnp.zeros((16,), jnp.float32)

        def zrow(r, carry):
            for j in range(d // 16):
                rows[2][r, pl.ds(j * 16, 16)] = zero
            return carry

        lax.fori_loop(0, _CHUNK, zrow, 0)

        def idx_start(j, ib):
            pltpu.async_copy(
                src_hbm.at[wid, j, 0],
                sidx.at[pl.ds(ib * _CHUNK, _CHUNK)], isem[ib])
            pltpu.async_copy(dst_hbm.at[wid, j, 0], didx.at[8 * ib],
                             isem[ib])

        def idx_wait(j, ib):
            pltpu.make_async_copy(
                src_hbm.at[wid, j, 0],
                sidx.at[pl.ds(ib * _CHUNK, _CHUNK)], isem[ib]).wait()
            pltpu.make_async_copy(
                dst_hbm.at[wid, j, 0], didx.at[8 * ib], isem[ib]).wait()

        def gather_start(b, ib):
            pltpu.async_copy(
                x_hbm.at[sidx.at[pl.ds(ib * _CHUNK, _CHUNK)]], rows[b],
                gsem[b])

        def gather_wait(b, ib):
            pltpu.make_async_copy(
                x_hbm.at[sidx.at[pl.ds(ib * _CHUNK, _CHUNK)]], rows[b],
                gsem[b]).wait()

        def scatter_start(b, ib):
            pltpu.async_copy(rows[b], acc.at[didx.at[8 * ib]], ssem[b],
                             add=True)

        def scatter_wait(b, ib):
            pltpu.make_async_copy(rows[b], acc.at[didx.at[8 * ib]],
                                  ssem[b]).wait()

        def emit(j, b, ib, swait_prev=True, idx_pf=True, g_pf=True):
            gather_wait(b, ib)
            scatter_start(b, ib)
            if swait_prev:
                scatter_wait((b + 2) % _NBUF, (ib + _NIDX - 2) % _NIDX)
            if idx_pf:
                idx_start(j + 3, (ib + 3) % _NIDX)
            if g_pf:
                idx_wait(j + 2, (ib + 2) % _NIDX)
                gather_start((b + 2) % _NBUF, (ib + 2) % _NIDX)

        idx_start(0, 0)
        idx_start(1, 1)
        idx_start(2, 2)
        idx_wait(0, 0)
        gather_start(0, 0)
        idx_wait(1, 1)
        gather_start(1, 1)

        nz = jnp.where(s < _NWCH - _NS * (_NWCH // _NS),
                       _NWCH // _NS + 1, _NWCH // _NS)

        def zloop(i, carry):
            pltpu.sync_copy(
                rows[2], acc.at[pl.ds((s + i * _NS) * _CHUNK, _CHUNK)])
            return carry

        lax.fori_loop(0, nz, zloop, 0)
        plsc.subcore_barrier()

        emit(0, 0, 0, swait_prev=False)
        emit(1, 1, 1, swait_prev=False)

        def body(kk, carry):
            j0 = 2 + 8 * kk
            for u in range(8):
                emit(j0 + u, (2 + u) % _NBUF, (2 + u) % _NIDX)
            return carry

        lax.fori_loop(0, (_NCHUNK - 5) // 8, body, 0)

        emit(_NCHUNK - 3, (_NCHUNK - 3) % _NBUF, (_NCHUNK - 3) % _NIDX,
             idx_pf=False)
        emit(_NCHUNK - 2, (_NCHUNK - 2) % _NBUF, (_NCHUNK - 2) % _NIDX,
             idx_pf=False, g_pf=False)
        emit(_NCHUNK - 1, (_NCHUNK - 1) % _NBUF, (_NCHUNK - 1) % _NIDX,
             idx_pf=False, g_pf=False)
        scatter_wait((_NCHUNK - 2) % _NBUF, (_NCHUNK - 2) % _NIDX)
        scatter_wait((_NCHUNK - 1) % _NBUF, (_NCHUNK - 1) % _NIDX)
        plsc.subcore_barrier()

        def wloop(i, carry):
            r0 = (s + i * _NS) * _WROWS
            pltpu.sync_copy(
                acc.at[pl.ds(r0, _WROWS)],
                out_hbm.at[c, pl.ds(r0, _WROWS)],
            )
            return carry

        lax.fori_loop(0, nz, wloop, 0)

    return k(x, src, dst)


_BLK = 2000


def _root_matmul_tc(x, w_root, b):
    d = x.shape[1]
    h = w_root.shape[1]

    def body(x_ref, wr_ref, b_ref, o_ref):
        o_ref[...] = jnp.dot(
            x_ref[...], wr_ref[...],
            preferred_element_type=jnp.float32) + b_ref[...]

    return pl.pallas_call(
        body,
        grid=(_N // _BLK,),
        in_specs=[
            pl.BlockSpec((_BLK, d), lambda i: (i, 0)),
            pl.BlockSpec((d, h), lambda i: (0, 0)),
            pl.BlockSpec((1, h), lambda i: (0, 0)),
        ],
        out_specs=pl.BlockSpec((_BLK, h), lambda i: (i, 0)),
        out_shape=jax.ShapeDtypeStruct((_N, h), jnp.float32),
    )(x, w_root, b.reshape(1, h))


def _combine_tc(r, p, w_nei):
    d = p.shape[2]
    h = w_nei.shape[1]

    def body(r_ref, p_ref, wn_ref, o_ref):
        agg = p_ref[0] + p_ref[1]
        acc = r_ref[...] + jnp.dot(agg, wn_ref[...],
                                   preferred_element_type=jnp.float32)
        o_ref[...] = jnp.maximum(acc, 0.0)

    return pl.pallas_call(
        body,
        grid=(_N // _BLK,),
        in_specs=[
            pl.BlockSpec((_BLK, h), lambda i: (i, 0)),
            pl.BlockSpec((_NC, _BLK, d), lambda i: (0, i, 0)),
            pl.BlockSpec((d, h), lambda i: (0, 0)),
        ],
        out_specs=pl.BlockSpec((_BLK, h), lambda i: (i, 0)),
        out_shape=jax.ShapeDtypeStruct((_N, h), jnp.float32),
    )(r, p, w_nei)


def _combine2_pool_tc(r, p, w_nei, batch):
    d = p.shape[2]
    h = w_nei.shape[1]
    nblk = _N // _BLK

    def body(r_ref, p_ref, wn_ref, bat_ref, o_ref, acc_ref, cnt_ref):
        i = pl.program_id(0)
        agg = p_ref[0] + p_ref[1]
        hh = r_ref[...] + jnp.dot(agg, wn_ref[...],
                                  preferred_element_type=jnp.float32)
        hh = jnp.maximum(hh, 0.0)

        onehot = (bat_ref[...] ==
                  lax.broadcasted_iota(jnp.int32, (_BLK, _G), 1)
                  ).astype(jnp.float32)
        part = lax.dot_general(onehot, hh, (((0,), (0,)), ((), ())),
                               preferred_element_type=jnp.float32)
        ones = jnp.ones((_BLK, h), jnp.float32)
        pcnt = lax.dot_general(onehot, ones, (((0,), (0,)), ((), ())),
                               preferred_element_type=jnp.float32)

        @pl.when(i == 0)
        def _():
            acc_ref[...] = jnp.zeros_like(acc_ref)
            cnt_ref[...] = jnp.zeros_like(cnt_ref)

        acc_ref[...] += part
        cnt_ref[...] += pcnt

        @pl.when(i == nblk - 1)
        def _():
            o_ref[...] = acc_ref[...] / jnp.maximum(cnt_ref[...], 1.0)

    return pl.pallas_call(
        body,
        grid=(nblk,),
        in_specs=[
            pl.BlockSpec((_BLK, h), lambda i: (i, 0)),
            pl.BlockSpec((_NC, _BLK, d), lambda i: (0, i, 0)),
            pl.BlockSpec((d, h), lambda i: (0, 0)),
            pl.BlockSpec((_BLK, 1), lambda i: (i, 0)),
        ],
        out_specs=pl.BlockSpec((_G, h), lambda i: (0, 0)),
        out_shape=jax.ShapeDtypeStruct((_G, h), jnp.float32),
        scratch_shapes=[
            pltpu.VMEM((_G, h), jnp.float32),
            pltpu.VMEM((_G, h), jnp.float32),
        ],
    )(r, p, w_nei, batch.reshape(_N, 1))


def kernel(x, edge_index, batch, W1_root, W1_nei, b1, W2_root, W2_nei, b2):
    src = edge_index[0].reshape(_NW, _NCHUNK, 1, _CHUNK)
    dst = edge_index[1].reshape(_NW, _NCHUNK, 1, _CHUNK)
    p1 = _segment_sum_sc(x, src, dst)
    r1 = _root_matmul_tc(x, W1_root, b1)
    h = _combine_tc(r1, p1, W1_nei)
    p2 = _segment_sum_sc(h, src, dst)
    r2 = _root_matmul_tc(h, W2_root, b2)
    return _combine2_pool_tc(r2, p2, W2_nei, batch)


# speedup vs baseline: 2.0623x; 1.0086x over previous
import functools

import jax
import jax.numpy as jnp
from jax import lax
from jax.experimental import pallas as pl
from jax.experimental.pallas import tpu as pltpu
from jax.experimental.pallas import tpu_sc as plsc

_N = 10000
_E = 320000
_D = 128
_G = 64

_NC = 2
_NS = 16
_NW = _NC * _NS
_EPT = _E // _NW
_CHUNK = 80
_NCHUNK = _EPT // _CHUNK
_WROWS = 80
_NWCH = _N // _WROWS


_NBUF = 4
_NIDX = 8


def _segment_sum_sc(x, src, dst):
    d = x.shape[1]
    mesh = plsc.VectorSubcoreMesh(core_axis_name="c", subcore_axis_name="s")

    @functools.partial(
        pl.kernel,
        out_type=jax.ShapeDtypeStruct((_NC, _N, d), jnp.float32),
        mesh=mesh,
        scratch_types=[
            pltpu.VMEM((_NIDX * _CHUNK,), jnp.int32),
            pltpu.VMEM((_NIDX * 8, _CHUNK), jnp.int32),
            [pltpu.VMEM((_CHUNK, d), jnp.float32) for _ in range(_NBUF)],
            pltpu.VMEM_SHARED((_N, d), jnp.float32),
            [pltpu.SemaphoreType.DMA for _ in range(_NIDX + 2 * _NBUF)],
        ],
    )
    def k(x_hbm, src_hbm, dst_hbm, out_hbm, sidx, didx, rows, acc, sems):
        c = lax.axis_index("c")
        s = lax.axis_index("s")
        wid = c * _NS + s
        isem = sems[:_NIDX]
        gsem = sems[_NIDX:_NIDX + _NBUF]
        ssem = sems[_NIDX + _NBUF:]

        zero = jnp.zeros((16,), jnp.float32)

        def zrow(r, carry):
            for j in range(d // 16):
                rows[2][r, pl.ds(j * 16, 16)] = zero
            return carry

        lax.fori_loop(0, _CHUNK, zrow, 0)

        def idx_start(j, ib):
            pltpu.async_copy(
                src_hbm.at[wid, j, 0],
                sidx.at[pl.ds(ib * _CHUNK, _CHUNK)], isem[ib])
            pltpu.async_copy(dst_hbm.at[wid, j, 0], didx.at[8 * ib],
                             isem[ib])

        def idx_wait(j, ib):
            pltpu.make_async_copy(
                src_hbm.at[wid, j, 0],
                sidx.at[pl.ds(ib * _CHUNK, _CHUNK)], isem[ib]).wait()
            pltpu.make_async_copy(
                dst_hbm.at[wid, j, 0], didx.at[8 * ib], isem[ib]).wait()

        def gather_start(b, ib):
            pltpu.async_copy(
                x_hbm.at[sidx.at[pl.ds(ib * _CHUNK, _CHUNK)]], rows[b],
                gsem[b])

        def gather_wait(b, ib):
            pltpu.make_async_copy(
                x_hbm.at[sidx.at[pl.ds(ib * _CHUNK, _CHUNK)]], rows[b],
                gsem[b]).wait()

        def scatter_start(b, ib):
            pltpu.async_copy(rows[b], acc.at[didx.at[8 * ib]], ssem[b],
                             add=True)

        def scatter_wait(b, ib):
            pltpu.make_async_copy(rows[b], acc.at[didx.at[8 * ib]],
                                  ssem[b]).wait()

        def emit(j, b, ib, swait_prev=True, idx_pf=True, g_pf=True):
            gather_wait(b, ib)
            scatter_start(b, ib)
            if swait_prev:
                scatter_wait((b + 2) % _NBUF, (ib + _NIDX - 2) % _NIDX)
            if idx_pf:
                idx_start(j + 3, (ib + 3) % _NIDX)
            if g_pf:
                idx_wait(j + 2, (ib + 2) % _NIDX)
                gather_start((b + 2) % _NBUF, (ib + 2) % _NIDX)

        idx_start(0, 0)
        idx_start(1, 1)
        idx_start(2, 2)
        idx_wait(0, 0)
        gather_start(0, 0)
        idx_wait(1, 1)
        gather_start(1, 1)

        nz = jnp.where(s < _NWCH - _NS * (_NWCH // _NS),
                       _NWCH // _NS + 1, _NWCH // _NS)

        def zloop(i, carry):
            pltpu.sync_copy(
                rows[2], acc.at[pl.ds((s + i * _NS) * _CHUNK, _CHUNK)])
            return carry

        lax.fori_loop(0, nz, zloop, 0)
        plsc.subcore_barrier()

        emit(0, 0, 0, swait_prev=False)
        emit(1, 1, 1, swait_prev=False)

        def body(kk, carry):
            j0 = 2 + 8 * kk
            for u in range(8):
                emit(j0 + u, (2 + u) % _NBUF, (2 + u) % _NIDX)
            return carry

        lax.fori_loop(0, (_NCHUNK - 5) // 8, body, 0)

        emit(_NCHUNK - 3, (_NCHUNK - 3) % _NBUF, (_NCHUNK - 3) % _NIDX,
             idx_pf=False)
        emit(_NCHUNK - 2, (_NCHUNK - 2) % _NBUF, (_NCHUNK - 2) % _NIDX,
             idx_pf=False, g_pf=False)
        emit(_NCHUNK - 1, (_NCHUNK - 1) % _NBUF, (_NCHUNK - 1) % _NIDX,
             idx_pf=False, g_pf=False)
        scatter_wait((_NCHUNK - 2) % _NBUF, (_NCHUNK - 2) % _NIDX)
        scatter_wait((_NCHUNK - 1) % _NBUF, (_NCHUNK - 1) % _NIDX)
        plsc.subcore_barrier()

        def wloop(i, carry):
            r0 = (s + i * _NS) * _WROWS
            pltpu.sync_copy(
                acc.at[pl.ds(r0, _WROWS)],
                out_hbm.at[c, pl.ds(r0, _WROWS)],
            )
            return carry

        lax.fori_loop(0, nz, wloop, 0)

    return k(x, src, dst)


_BLK = 2000


def _combine_tc(x, p, w_root, w_nei, b):
    d = p.shape[2]
    h = w_root.shape[1]

    def body(x_ref, p_ref, wr_ref, wn_ref, b_ref, o_ref):
        agg = p_ref[0] + p_ref[1]
        acc = jnp.dot(x_ref[...], wr_ref[...],
                      preferred_element_type=jnp.float32)
        acc = acc + jnp.dot(agg, wn_ref[...],
                            preferred_element_type=jnp.float32)
        o_ref[...] = jnp.maximum(acc + b_ref[...], 0.0)

    return pl.pallas_call(
        body,
        grid=(_N // _BLK,),
        in_specs=[
            pl.BlockSpec((_BLK, d), lambda i: (i, 0)),
            pl.BlockSpec((_NC, _BLK, d), lambda i: (0, i, 0)),
            pl.BlockSpec((d, h), lambda i: (0, 0)),
            pl.BlockSpec((d, h), lambda i: (0, 0)),
            pl.BlockSpec((1, h), lambda i: (0, 0)),
        ],
        out_specs=pl.BlockSpec((_BLK, h), lambda i: (i, 0)),
        out_shape=jax.ShapeDtypeStruct((_N, h), jnp.float32),
    )(x, p, w_root, w_nei, b.reshape(1, h))


def _combine2_pool_tc(x, p, w_root, w_nei, b, batch):
    d = p.shape[2]
    h = w_root.shape[1]
    nblk = _N // _BLK

    def body(x_ref, p_ref, wr_ref, wn_ref, b_ref, bat_ref, o_ref,
             acc_ref, cnt_ref):
        i = pl.program_id(0)
        agg = p_ref[0] + p_ref[1]
        hh = jnp.dot(x_ref[...], wr_ref[...],
                     preferred_element_type=jnp.float32)
        hh = hh + jnp.dot(agg, wn_ref[...],
                          preferred_element_type=jnp.float32)
        hh = jnp.maximum(hh + b_ref[...], 0.0)

        onehot = (bat_ref[...] ==
                  lax.broadcasted_iota(jnp.int32, (_BLK, _G), 1)
                  ).astype(jnp.float32)
        part = lax.dot_general(onehot, hh, (((0,), (0,)), ((), ())),
                               preferred_element_type=jnp.float32)
        ones = jnp.ones((_BLK, h), jnp.float32)
        pcnt = lax.dot_general(onehot, ones, (((0,), (0,)), ((), ())),
                               preferred_element_type=jnp.float32)

        @pl.when(i == 0)
        def _():
            acc_ref[...] = jnp.zeros_like(acc_ref)
            cnt_ref[...] = jnp.zeros_like(cnt_ref)

        acc_ref[...] += part
        cnt_ref[...] += pcnt

        @pl.when(i == nblk - 1)
        def _():
            o_ref[...] = acc_ref[...] / jnp.maximum(cnt_ref[...], 1.0)

    return pl.pallas_call(
        body,
        grid=(nblk,),
        in_specs=[
            pl.BlockSpec((_BLK, d), lambda i: (i, 0)),
            pl.BlockSpec((_NC, _BLK, d), lambda i: (0, i, 0)),
            pl.BlockSpec((d, h), lambda i: (0, 0)),
            pl.BlockSpec((d, h), lambda i: (0, 0)),
            pl.BlockSpec((1, h), lambda i: (0, 0)),
            pl.BlockSpec((_BLK, 1), lambda i: (i, 0)),
        ],
        out_specs=pl.BlockSpec((_G, h), lambda i: (0, 0)),
        out_shape=jax.ShapeDtypeStruct((_G, h), jnp.float32),
        scratch_shapes=[
            pltpu.VMEM((_G, h), jnp.float32),
            pltpu.VMEM((_G, h), jnp.float32),
        ],
    )(x, p, w_root, w_nei, b.reshape(1, h), batch.reshape(_N, 1))


def kernel(x, edge_index, batch, W1_root, W1_nei, b1, W2_root, W2_nei, b2):
    src = edge_index[0].reshape(_NW, _NCHUNK, 1, _CHUNK)
    dst = edge_index[1].reshape(_NW, _NCHUNK, 1, _CHUNK)
    p1 = _segment_sum_sc(x, src, dst)
    h = _combine_tc(x, p1, W1_root, W1_nei, b1)
    p2 = _segment_sum_sc(h, src, dst)
    return _combine2_pool_tc(h, p2, W2_root, W2_nei, b2, batch)
